# baseline (speedup 1.0000x reference)
import functools

import jax
import jax.numpy as jnp
from jax import lax
from jax.experimental import pallas as pl
from jax.experimental.pallas import tpu as pltpu
from jax.experimental.pallas import tpu_sc as plsc

NW = 32
CHUNK = 128
NBUF = 5
PF = 3


def _wait(src, dst, sem):
    pltpu.make_async_copy(src, dst, sem).wait()


@functools.lru_cache(maxsize=None)
def _build(n_chunk, n_rows, d):
    mesh = plsc.VectorSubcoreMesh(core_axis_name="c", subcore_axis_name="s")
    n_blk = n_chunk // NBUF

    @functools.partial(
        pl.kernel,
        mesh=mesh,
        out_type=jax.ShapeDtypeStruct((NW * n_chunk * CHUNK, d), jnp.float32),
        scratch_types=[
            pltpu.VMEM((n_chunk * CHUNK,), jnp.int32),
            pltpu.VMEM((NBUF, CHUNK, d), jnp.float32),
            pltpu.SemaphoreType.DMA((NBUF,)),
            pltpu.SemaphoreType.DMA((NBUF,)),
        ],
        compiler_params=pltpu.CompilerParams(use_tc_tiling_on_sc=True),
    )
    def emb(ids_hbm, table_hbm, out_hbm, idx_v, rows_v, gsem, ssem):
        wid = lax.axis_index("s") * 2 + lax.axis_index("c")
        base = wid * (n_chunk * CHUNK)
        pltpu.sync_copy(ids_hbm.at[pl.ds(base, n_chunk * CHUNK)], idx_v)

        def gather(j, b):
            pltpu.async_copy(
                table_hbm.at[idx_v.at[pl.ds(j * CHUNK, CHUNK)]],
                rows_v.at[b],
                gsem.at[b],
            )

        def wait_gather(b):
            _wait(table_hbm.at[pl.ds(0, CHUNK)], rows_v.at[b], gsem.at[b])

        def store(j, b):
            pltpu.async_copy(
                rows_v.at[b], out_hbm.at[pl.ds(base + j * CHUNK, CHUNK)], ssem.at[b]
            )

        def wait_store(b):
            _wait(rows_v.at[b], out_hbm.at[pl.ds(base, CHUNK)], ssem.at[b])

        for j in range(PF):
            gather(j, j)

        for b in range(NBUF):
            wait_gather(b)
            store(b, b)
            bn = (b + PF) % NBUF
            if b + PF < NBUF:
                gather(b + PF, bn)
            else:
                wait_store(bn)
                gather(b + PF, bn)

        def body(g, carry):
            j0 = g * NBUF
            for b in range(NBUF):
                wait_gather(b)
                store(j0 + b, b)
                bn = (b + PF) % NBUF
                wait_store(bn)
                gather(j0 + b + PF, bn)
            return carry

        lax.fori_loop(1, n_blk - 1, body, 0)

        j0 = (n_blk - 1) * NBUF
        for b in range(NBUF):
            wait_gather(b)
            store(j0 + b, b)
            if b + PF < NBUF:
                bn = (b + PF) % NBUF
                wait_store(bn)
                gather(j0 + b + PF, bn)
        for b in range(NBUF):
            wait_store(b)

    return emb


def kernel(token_ids, weight):
    b, s = token_ids.shape
    total = b * s
    n_chunk = total // (NW * CHUNK)
    d = weight.shape[1]
    ids = token_ids.T.reshape(total).astype(jnp.int32)
    out = _build(n_chunk, weight.shape[0], d)(ids, weight)
    return out.reshape(s, b, d).transpose(1, 0, 2)


# speedup vs baseline: 1.0033x; 1.0033x over previous
import functools

import jax
import jax.numpy as jnp
from jax import lax
from jax.experimental import pallas as pl
from jax.experimental.pallas import tpu as pltpu
from jax.experimental.pallas import tpu_sc as plsc

NW = 32
CHUNK = 128
NBUF = 5
PF = 3


def _wait(src, dst, sem):
    pltpu.make_async_copy(src, dst, sem).wait()


@functools.lru_cache(maxsize=None)
def _build(n_chunk, n_rows, d):
    mesh = plsc.VectorSubcoreMesh(core_axis_name="c", subcore_axis_name="s")
    n_blk = n_chunk // NBUF

    @functools.partial(
        pl.kernel,
        mesh=mesh,
        out_type=jax.ShapeDtypeStruct((NW * n_chunk * CHUNK, d), jnp.float32),
        scratch_types=[
            pltpu.VMEM((n_chunk * CHUNK,), jnp.int32),
            pltpu.VMEM((NBUF, CHUNK, d), jnp.float32),
            pltpu.SemaphoreType.DMA((NBUF,)),
            pltpu.SemaphoreType.DMA((NBUF,)),
        ],
        compiler_params=pltpu.CompilerParams(
            use_tc_tiling_on_sc=True,
            disable_bounds_checks=True,
            skip_device_barrier=True,
        ),
    )
    def emb(ids_hbm, table_hbm, out_hbm, idx_v, rows_v, gsem, ssem):
        wid = lax.axis_index("s") * 2 + lax.axis_index("c")
        base = wid * (n_chunk * CHUNK)
        pltpu.sync_copy(ids_hbm.at[pl.ds(base, n_chunk * CHUNK)], idx_v)

        def gather(j, b):
            pltpu.async_copy(
                table_hbm.at[idx_v.at[pl.ds(j * CHUNK, CHUNK)]],
                rows_v.at[b],
                gsem.at[b],
            )

        def wait_gather(b):
            _wait(table_hbm.at[pl.ds(0, CHUNK)], rows_v.at[b], gsem.at[b])

        def store(j, b):
            pltpu.async_copy(
                rows_v.at[b], out_hbm.at[pl.ds(base + j * CHUNK, CHUNK)], ssem.at[b]
            )

        def wait_store(b):
            _wait(rows_v.at[b], out_hbm.at[pl.ds(base, CHUNK)], ssem.at[b])

        for j in range(PF):
            gather(j, j)

        for b in range(NBUF):
            wait_gather(b)
            store(b, b)
            bn = (b + PF) % NBUF
            if b + PF < NBUF:
                gather(b + PF, bn)
            else:
                wait_store(bn)
                gather(b + PF, bn)

        def body(g, carry):
            j0 = g * NBUF
            for b in range(NBUF):
                wait_gather(b)
                store(j0 + b, b)
                bn = (b + PF) % NBUF
                wait_store(bn)
                gather(j0 + b + PF, bn)
            return carry

        lax.fori_loop(1, n_blk - 1, body, 0)

        j0 = (n_blk - 1) * NBUF
        for b in range(NBUF):
            wait_gather(b)
            store(j0 + b, b)
            if b + PF < NBUF:
                bn = (b + PF) % NBUF
                wait_store(bn)
                gather(j0 + b + PF, bn)
        for b in range(NBUF):
            wait_store(b)

    return emb


def kernel(token_ids, weight):
    b, s = token_ids.shape
    total = b * s
    n_chunk = total // (NW * CHUNK)
    d = weight.shape[1]
    ids = token_ids.T.reshape(total).astype(jnp.int32)
    out = _build(n_chunk, weight.shape[0], d)(ids, weight)
    return out.reshape(s, b, d).transpose(1, 0, 2)
